# baseline (speedup 1.0000x reference)
import functools

import jax
import jax.numpy as jnp
from jax import lax
from jax.experimental import pallas as pl
from jax.experimental.pallas import tpu as pltpu
from jax.experimental.pallas import tpu_sc as plsc

_D = 64
_B = 16384
_NC, _NS, _L = 2, 16, 16
_NW = _NC * _NS
_BPW = _B // _NW
_CH = 128
_NCH = _BPW // _CH

_mesh = plsc.VectorSubcoreMesh(core_axis_name="c", subcore_axis_name="s")
_params = pltpu.CompilerParams(
    needs_layout_passes=False, use_tc_tiling_on_sc=False)


@functools.partial(
    pl.kernel,
    mesh=_mesh,
    out_type=jax.ShapeDtypeStruct((_B, _D), jnp.float32),
    compiler_params=_params,
    scratch_types=[
        pltpu.VMEM((_NCH, _CH), jnp.int32),
        pltpu.VMEM((_BPW, _D), jnp.float32),
        pltpu.SemaphoreType.DMA,
    ],
)
def _gather(idx_h, tbl_h, rows_h, idx_v, rows_v, sem):
    wid = lax.axis_index("s") * _NC + lax.axis_index("c")
    base = wid * _BPW

    pltpu.sync_copy(idx_h.at[pl.ds(wid * _NCH, _NCH)], idx_v)
    copies = [
        pltpu.async_copy(tbl_h.at[idx_v.at[j]],
                         rows_v.at[pl.ds(j * _CH, _CH)], sem)
        for j in range(_NCH)
    ]
    for c in copies:
        c.wait()
    pltpu.sync_copy(rows_v, rows_h.at[pl.ds(base, _BPW)])


@functools.partial(
    pl.kernel,
    mesh=_mesh,
    out_type=jax.ShapeDtypeStruct((_B,), jnp.float32),
    compiler_params=_params,
    scratch_types=[
        pltpu.VMEM((_BPW, _D), jnp.float32),
        pltpu.VMEM((_BPW, _D), jnp.float32),
        pltpu.VMEM((_BPW,), jnp.float32),
    ],
)
def _dot(vrows_h, urows_h, o_h, vbuf, ubuf, obuf):
    wid = lax.axis_index("s") * _NC + lax.axis_index("c")
    base = wid * _BPW

    pltpu.sync_copy(vrows_h.at[pl.ds(base, _BPW)], vbuf)
    pltpu.sync_copy(urows_h.at[pl.ds(base, _BPW)], ubuf)

    iota = lax.iota(jnp.int32, _L)
    last = jnp.full((_L,), _L - 1, jnp.int32)

    def group_body(g, carry):
        outv = jnp.zeros((_L,), jnp.float32)
        for k in range(_L):
            b = g * _L + k
            acc = vbuf[b, pl.ds(0, _L)] * ubuf[b, pl.ds(0, _L)]
            for c in range(1, _D // _L):
                acc = acc + (vbuf[b, pl.ds(c * _L, _L)]
                             * ubuf[b, pl.ds(c * _L, _L)])
            tot = jnp.cumsum(acc)
            bcast = tot.at[last].get(mode="promise_in_bounds")
            outv = jnp.where(iota == k, bcast, outv)
        obuf[pl.ds(g * _L, _L)] = outv
        return carry

    lax.fori_loop(0, _BPW // _L, group_body, 0)

    pltpu.sync_copy(obuf, o_h.at[pl.ds(base, _BPW)])


def kernel(center, context, in_embed, out_embed):
    c2 = center.astype(jnp.int32).reshape(_NW * _NCH, _CH)
    x2 = context.astype(jnp.int32).reshape(_NW * _NCH, _CH)
    v = _gather(c2, in_embed)
    u = _gather(x2, out_embed)
    return _dot(v, u)


# speedup vs baseline: 1.2684x; 1.2684x over previous
import functools

import jax
import jax.numpy as jnp
from jax import lax
from jax.experimental import pallas as pl
from jax.experimental.pallas import tpu as pltpu
from jax.experimental.pallas import tpu_sc as plsc

_D = 64
_B = 16384
_NC, _NS, _L = 2, 16, 16
_NW = _NC * _NS
_BPW = _B // _NW
_CH = 128
_NCH = _BPW // _CH

_mesh = plsc.VectorSubcoreMesh(core_axis_name="c", subcore_axis_name="s")


@functools.partial(
    pl.kernel,
    mesh=_mesh,
    out_type=jax.ShapeDtypeStruct((_B // 2, 2 * _D), jnp.float32),
    compiler_params=pltpu.CompilerParams(
        needs_layout_passes=False, use_tc_tiling_on_sc=True),
    scratch_types=[
        pltpu.VMEM((_BPW,), jnp.int32),
        pltpu.VMEM((_BPW // 2, 2 * _D), jnp.float32),
        pltpu.SemaphoreType.DMA,
    ],
)
def _gather_native(idx_h, tbl_h, rows_h, idx_v, rows_v, sem):
    wid = lax.axis_index("s") * _NC + lax.axis_index("c")
    base = wid * _BPW

    pltpu.sync_copy(idx_h.at[pl.ds(base, _BPW)], idx_v)

    def issue_body(g, carry):
        vec = idx_v[pl.ds(g * _L, _L)]
        for k in range(_L):
            p = g * (_L // 2) + k // 2
            off = (k % 2) * _D
            pltpu.async_copy(tbl_h.at[vec[k]],
                             rows_v.at[p, pl.ds(off, _D)], sem)
        return carry

    lax.fori_loop(0, _BPW // _L, issue_body, 0)
    pltpu.make_async_copy(tbl_h.at[pl.ds(0, _BPW // 2)], rows_v, sem).wait()
    pltpu.sync_copy(rows_v, rows_h.at[pl.ds(wid * (_BPW // 2), _BPW // 2)])


@functools.partial(
    pl.kernel,
    mesh=_mesh,
    out_type=jax.ShapeDtypeStruct((_B,), jnp.float32),
    compiler_params=pltpu.CompilerParams(
        needs_layout_passes=False, use_tc_tiling_on_sc=False),
    scratch_types=[
        pltpu.VMEM((_NCH, _CH), jnp.int32),
        pltpu.VMEM((_BPW, _D), jnp.float32),
        pltpu.VMEM((_BPW // 2, 2 * _D), jnp.float32),
        pltpu.VMEM((_BPW,), jnp.float32),
        pltpu.SemaphoreType.DMA,
    ],
)
def _gather_dot(xidx_h, utbl_h, vrows_h, o_h, xidx_v, ubuf, vbuf, obuf, sem):
    wid = lax.axis_index("s") * _NC + lax.axis_index("c")
    base = wid * _BPW

    pltpu.sync_copy(xidx_h.at[pl.ds(wid * _NCH, _NCH)], xidx_v)
    copies = [
        pltpu.async_copy(utbl_h.at[xidx_v.at[j]],
                         ubuf.at[pl.ds(j * _CH, _CH)], sem)
        for j in range(_NCH)
    ]
    pltpu.sync_copy(vrows_h.at[pl.ds(wid * (_BPW // 2), _BPW // 2)], vbuf)
    for c in copies:
        c.wait()

    iota = lax.iota(jnp.int32, _L)
    last = jnp.full((_L,), _L - 1, jnp.int32)

    def group_body(g, carry):
        outv = jnp.zeros((_L,), jnp.float32)
        for k in range(_L):
            b = g * _L + k
            p = g * (_L // 2) + k // 2
            off = (k % 2) * _D
            acc = vbuf[p, pl.ds(off, _L)] * ubuf[b, pl.ds(0, _L)]
            for c in range(1, _D // _L):
                acc = acc + (vbuf[p, pl.ds(off + c * _L, _L)]
                             * ubuf[b, pl.ds(c * _L, _L)])
            tot = jnp.cumsum(acc)
            bcast = tot.at[last].get(mode="promise_in_bounds")
            outv = jnp.where(iota == k, bcast, outv)
        obuf[pl.ds(g * _L, _L)] = outv
        return carry

    lax.fori_loop(0, _BPW // _L, group_body, 0)

    pltpu.sync_copy(obuf, o_h.at[pl.ds(base, _BPW)])


def kernel(center, context, in_embed, out_embed):
    x2 = context.astype(jnp.int32).reshape(_NW * _NCH, _CH)
    v = _gather_native(center.astype(jnp.int32), in_embed)
    return _gather_dot(x2, out_embed, v)
